# baseline (speedup 1.0000x reference)
import functools

import jax
import jax.numpy as jnp
from jax import lax
from jax.experimental import pallas as pl
from jax.experimental.pallas import tpu as pltpu
from jax.experimental.pallas import tpu_sc as plsc

_N = 2
_DHW = 128
_M = _N * _DHW ** 3
_P = _N * _DHW ** 3
_NC, _NS, _L = 2, 16, 16
_PW = _DHW ** 3 // _NS
_B = 2048
_NG = _B // _L
_NCH = _PW // _B
_NU = _NCH // 2

_S = 65
_SV = _S * _S * _S
_NSR = 16
_SPAN = 16 * _DHW + 80
_BUF = 2176
_RR = 1056
_RV = 1040

_OFFS = [dz * _DHW * _DHW + dy * _DHW + dx
         for dz in (0, 1) for dy in (0, 1) for dx in (0, 1)]

_mesh = plsc.VectorSubcoreMesh(
    core_axis_name="c", subcore_axis_name="s", num_cores=_NC, num_subcores=_NS
)


@functools.partial(
    pl.kernel,
    out_type=jax.ShapeDtypeStruct((_P,), jnp.float32),
    mesh=_mesh,
    scratch_types=[
        pltpu.HBM((_N * _SV, 8), jnp.float32),
        pltpu.VMEM((_BUF,), jnp.float32),
        pltpu.VMEM((_BUF,), jnp.float32),
        pltpu.VMEM((_BUF,), jnp.float32),
        pltpu.VMEM((_BUF,), jnp.float32),
        pltpu.VMEM((_RR, 8), jnp.float32),
        pltpu.VMEM((_RR, 8), jnp.float32),
        pltpu.VMEM((_B,), jnp.float32),
        pltpu.VMEM((_B,), jnp.float32),
        pltpu.VMEM((_B,), jnp.float32),
        pltpu.VMEM((_B,), jnp.float32),
        pltpu.VMEM((_B,), jnp.float32),
        pltpu.VMEM((_B,), jnp.float32),
        pltpu.VMEM((_B,), jnp.int32),
        pltpu.VMEM((_B,), jnp.int32),
        pltpu.VMEM((_B, 8), jnp.float32),
        pltpu.VMEM((_B, 8), jnp.float32),
        pltpu.VMEM((_B,), jnp.float32),
        pltpu.VMEM((_B,), jnp.float32),
        pltpu.VMEM((_B,), jnp.float32),
        pltpu.VMEM((_B,), jnp.float32),
        pltpu.VMEM((_B,), jnp.float32),
        pltpu.VMEM((_B,), jnp.float32),
        pltpu.VMEM((_B,), jnp.float32),
        pltpu.VMEM((_B,), jnp.float32),
        pltpu.SemaphoreType.DMA,
        pltpu.SemaphoreType.DMA,
        pltpu.SemaphoreType.DMA,
        pltpu.SemaphoreType.DMA,
        pltpu.SemaphoreType.DMA,
        pltpu.SemaphoreType.DMA,
        pltpu.SemaphoreType.DMA,
    ],
    compiler_params=pltpu.CompilerParams(
        needs_layout_passes=False, use_tc_tiling_on_sc=False
    ),
)
def _sample_kernel(gx_hbm, gy_hbm, gz_hbm, src_hbm, out_hbm, tabs_hbm,
                   b0a, b1a, b0b, b1b, rba, rbb,
                   gx0, gy0, gz0, gx1, gy1, gz1, ix0, ix1, v80, v81,
                   fx0, fy0, fz0, fx1, fy1, fz1, ob0, ob1,
                   sem, semp, semo, sg0, sg1, so0, so1):
    c = lax.axis_index("c")
    s = lax.axis_index("s")
    io = lax.iota(jnp.int32, _L)
    half = jnp.float32(0.5 * (_DHW - 1))
    nsrc = c * (_DHW ** 3)
    trow0 = c * _SV

    kcols = [jnp.full((_L,), k, jnp.int32) for k in range(8)]

    def slab_start(t, dz):
        sr = s * _NSR + t
        zi = sr // 4
        y0 = (sr % 4) * 16
        return nsrc + (63 + zi + dz) * (_DHW * _DHW) + (63 + y0) * _DHW + 56

    def fire_slabs(t, b0, b1):
        cp0 = pltpu.async_copy(
            src_hbm.at[pl.ds(slab_start(t, 0), _SPAN)],
            b0.at[pl.ds(0, _SPAN)], semp)
        cp1 = pltpu.async_copy(
            src_hbm.at[pl.ds(slab_start(t, 1), _SPAN)],
            b1.at[pl.ds(0, _SPAN)], semp)
        return cp0, cp1

    def wait_slabs(b0, b1):
        pltpu.make_async_copy(
            src_hbm.at[pl.ds(0, _SPAN)], b0.at[pl.ds(0, _SPAN)], semp).wait()
        pltpu.make_async_copy(
            src_hbm.at[pl.ds(0, _SPAN)], b1.at[pl.ds(0, _SPAN)], semp).wait()

    def assemble(b0, b1, rb):
        def yrow(j, c1):
            off0 = j * _DHW + 7
            for gx in range(5):
                off = off0 + gx * _L
                rowv = j * _S + gx * _L + io
                vs = [b0[pl.ds(off, _L)], b0[pl.ds(off + 1, _L)],
                      b0[pl.ds(off + _DHW, _L)], b0[pl.ds(off + _DHW + 1, _L)],
                      b1[pl.ds(off, _L)], b1[pl.ds(off + 1, _L)],
                      b1[pl.ds(off + _DHW, _L)], b1[pl.ds(off + _DHW + 1, _L)]]
                for k in range(8):
                    plsc.store_scatter(rb, [rowv, kcols[k]], vs[k])
            return c1

        lax.fori_loop(0, 16, yrow, 0)

    def fire_rows(t, rb):
        sr = s * _NSR + t
        rbase = trow0 + (sr // 4) * (_S * _S) + (sr % 4) * _RV
        return pltpu.async_copy(
            rb.at[pl.ds(0, _RV)], tabs_hbm.at[pl.ds(rbase, _RV)], semo)

    def wait_rows(rb):
        pltpu.make_async_copy(
            rb.at[pl.ds(0, _RV)], tabs_hbm.at[pl.ds(0, _RV)], semo).wait()

    fire_slabs(0, b0a, b1a)

    def p1_body(u, carry1):
        ta = 2 * u
        tb = 2 * u + 1
        wait_slabs(b0a, b1a)
        fire_slabs(tb, b0b, b1b)

        @pl.when(u > 0)
        def _():
            wait_rows(rba)

        assemble(b0a, b1a, rba)
        fire_rows(ta, rba)

        wait_slabs(b0b, b1b)

        @pl.when(u < (_NSR // 2 - 1))
        def _():
            fire_slabs(ta + 2, b0a, b1a)

        @pl.when(u > 0)
        def _():
            wait_rows(rbb)

        assemble(b0b, b1b, rbb)
        fire_rows(tb, rbb)
        return carry1

    lax.fori_loop(0, _NSR // 2, p1_body, 0)
    wait_rows(rba)
    wait_rows(rbb)
    plsc.subcore_barrier()

    base_pt = c * (_DHW ** 3) + s * _PW
    rbias = trow0 - 63 * (_S * _S + _S + 1)

    def fire_grid(i, gxb, gyb, gzb, sg):
        sl_hbm = pl.ds(base_pt + i * _B, _B)
        pltpu.async_copy(gx_hbm.at[sl_hbm], gxb, sg)
        pltpu.async_copy(gy_hbm.at[sl_hbm], gyb, sg)
        pltpu.async_copy(gz_hbm.at[sl_hbm], gzb, sg)

    def wait_grid(gxb, gyb, gzb, sg):
        for b in (gxb, gyb, gzb):
            pltpu.make_async_copy(gx_hbm.at[pl.ds(0, _B)], b, sg).wait()

    def pass1(gxb, gyb, gzb, fxb, fyb, fzb, idxb):
        def one1(g):
            sl = pl.ds(g * _L, _L)
            px = (gxb[sl] + 1.0) * half
            py = (gyb[sl] + 1.0) * half
            pz = (gzb[sl] + 1.0) * half
            x0 = px.astype(jnp.int32)
            y0 = py.astype(jnp.int32)
            z0 = pz.astype(jnp.int32)
            fxb[sl] = px - x0.astype(jnp.float32)
            fyb[sl] = py - y0.astype(jnp.float32)
            fzb[sl] = pz - z0.astype(jnp.float32)
            idxb[sl] = z0 * (_S * _S) + y0 * _S + x0 + rbias

        def body1(g, c1):
            one1(2 * g)
            one1(2 * g + 1)
            return c1

        lax.fori_loop(0, _NG // 2, body1, 0)

    def fire_gather(idxb, vals8):
        pltpu.async_copy(tabs_hbm.at[idxb], vals8, sem)

    def wait_gather(idxb, vals8):
        pltpu.make_async_copy(tabs_hbm.at[idxb], vals8, sem).wait()

    def pass2(vals8, fxb, fyb, fzb, obuf):
        def one2(g):
            sl = pl.ds(g * _L, _L)
            pi = io + g * _L
            v = [plsc.load_gather(vals8, [pi, jnp.full((_L,), k, jnp.int32)])
                 for k in range(8)]
            fx = fxb[sl]
            fy = fyb[sl]
            fz = fzb[sl]
            a00 = v[0] + fx * (v[1] - v[0])
            a01 = v[2] + fx * (v[3] - v[2])
            a10 = v[4] + fx * (v[5] - v[4])
            a11 = v[6] + fx * (v[7] - v[6])
            b0 = a00 + fy * (a01 - a00)
            b1 = a10 + fy * (a11 - a10)
            obuf[sl] = b0 + fz * (b1 - b0)

        def body2(g, c2):
            one2(2 * g)
            one2(2 * g + 1)
            return c2

        lax.fori_loop(0, _NG // 2, body2, 0)

    def fire_out(i, obuf, so):
        pltpu.async_copy(obuf, out_hbm.at[pl.ds(base_pt + i * _B, _B)], so)

    def wait_out(obuf, so):
        pltpu.make_async_copy(obuf, out_hbm.at[pl.ds(0, _B)], so).wait()

    fire_grid(0, gx0, gy0, gz0, sg0)
    wait_grid(gx0, gy0, gz0, sg0)
    pass1(gx0, gy0, gz0, fx0, fy0, fz0, ix0)
    fire_gather(ix0, v80)
    fire_grid(1, gx1, gy1, gz1, sg1)

    def p2_body(u, carry2):
        a = 2 * u

        wait_grid(gx1, gy1, gz1, sg1)
        pass1(gx1, gy1, gz1, fx1, fy1, fz1, ix1)
        wait_gather(ix0, v80)
        fire_gather(ix1, v81)

        @pl.when(u > 0)
        def _():
            wait_out(ob0, so0)

        @pl.when(u < _NU - 1)
        def _():
            fire_grid(a + 2, gx0, gy0, gz0, sg0)

        pass2(v80, fx0, fy0, fz0, ob0)
        fire_out(a, ob0, so0)

        @pl.when(u < _NU - 1)
        def _():
            wait_grid(gx0, gy0, gz0, sg0)
            pass1(gx0, gy0, gz0, fx0, fy0, fz0, ix0)

        wait_gather(ix1, v81)

        @pl.when(u < _NU - 1)
        def _():
            fire_gather(ix0, v80)

        @pl.when(u > 0)
        def _():
            wait_out(ob1, so1)

        @pl.when(u < _NU - 1)
        def _():
            fire_grid(a + 3, gx1, gy1, gz1, sg1)

        pass2(v81, fx1, fy1, fz1, ob1)
        fire_out(a + 1, ob1, so1)
        return carry2

    lax.fori_loop(0, _NU, p2_body, 0)
    wait_out(ob0, so0)
    wait_out(ob1, so1)


def kernel(source, affine_grid):
    n, ch, d, h, w = source.shape
    assert (n, ch, d, h, w) == (_N, 1, _DHW, _DHW, _DHW)
    src_flat = jnp.pad(source.reshape(-1), (0, 128))
    gx = affine_grid[..., 0].reshape(-1)
    gy = affine_grid[..., 1].reshape(-1)
    gz = affine_grid[..., 2].reshape(-1)
    out = _sample_kernel(gx, gy, gz, src_flat)
    return out.reshape(n, ch, d, h, w)


# speedup vs baseline: 1.0361x; 1.0361x over previous
import functools

import jax
import jax.numpy as jnp
from jax import lax
from jax.experimental import pallas as pl
from jax.experimental.pallas import tpu as pltpu
from jax.experimental.pallas import tpu_sc as plsc

_N = 2
_DHW = 128
_M = _N * _DHW ** 3
_P = _N * _DHW ** 3
_NC, _NS, _L = 2, 16, 16
_PW = _DHW ** 3 // _NS
_B = 2048
_NG = _B // _L
_NCH = _PW // _B
_NU = _NCH // 2

_S = 65
_SV = _S * _S * _S
_NSR = 16
_SPAN = 16 * _DHW + 80
_BUF = 2176
_RR = 1056
_RV = 1040

_OFFS = [dz * _DHW * _DHW + dy * _DHW + dx
         for dz in (0, 1) for dy in (0, 1) for dx in (0, 1)]

_mesh = plsc.VectorSubcoreMesh(
    core_axis_name="c", subcore_axis_name="s", num_cores=_NC, num_subcores=_NS
)


@functools.partial(
    pl.kernel,
    out_type=jax.ShapeDtypeStruct((_P,), jnp.float32),
    mesh=_mesh,
    scratch_types=[
        pltpu.HBM((_N * _SV, 8), jnp.float32),
        pltpu.VMEM((_BUF,), jnp.float32),
        pltpu.VMEM((_BUF,), jnp.float32),
        pltpu.VMEM((_BUF,), jnp.float32),
        pltpu.VMEM((_BUF,), jnp.float32),
        pltpu.VMEM((_RR, 8), jnp.float32),
        pltpu.VMEM((_RR, 8), jnp.float32),
        pltpu.VMEM((_B,), jnp.float32),
        pltpu.VMEM((_B,), jnp.float32),
        pltpu.VMEM((_B,), jnp.float32),
        pltpu.VMEM((_B,), jnp.float32),
        pltpu.VMEM((_B,), jnp.float32),
        pltpu.VMEM((_B,), jnp.float32),
        pltpu.VMEM((_B,), jnp.int32),
        pltpu.VMEM((_B,), jnp.int32),
        pltpu.VMEM((_B, 8), jnp.float32),
        pltpu.VMEM((_B, 8), jnp.float32),
        pltpu.VMEM((_B,), jnp.float32),
        pltpu.VMEM((_B,), jnp.float32),
        pltpu.VMEM((_B,), jnp.float32),
        pltpu.VMEM((_B,), jnp.float32),
        pltpu.VMEM((_B,), jnp.float32),
        pltpu.VMEM((_B,), jnp.float32),
        pltpu.VMEM((_B,), jnp.float32),
        pltpu.VMEM((_B,), jnp.float32),
        pltpu.SemaphoreType.DMA,
        pltpu.SemaphoreType.DMA,
        pltpu.SemaphoreType.DMA,
        pltpu.SemaphoreType.DMA,
        pltpu.SemaphoreType.DMA,
        pltpu.SemaphoreType.DMA,
        pltpu.SemaphoreType.DMA,
    ],
    compiler_params=pltpu.CompilerParams(
        needs_layout_passes=False, use_tc_tiling_on_sc=False
    ),
)
def _sample_kernel(gx_hbm, gy_hbm, gz_hbm, src_hbm, out_hbm, tabs_hbm,
                   b0a, b1a, b0b, b1b, rba, rbb,
                   gx0, gy0, gz0, gx1, gy1, gz1, ix0, ix1, v80, v81,
                   fx0, fy0, fz0, fx1, fy1, fz1, ob0, ob1,
                   sem, semp, semo, sg0, sg1, so0, so1):
    c = lax.axis_index("c")
    s = lax.axis_index("s")
    io = lax.iota(jnp.int32, _L)
    half = jnp.float32(0.5 * (_DHW - 1))
    nsrc = c * (_DHW ** 3)
    trow0 = c * _SV

    kcols = [jnp.full((_L,), k, jnp.int32) for k in range(8)]

    def slab_start(t, dz):
        sr = s * _NSR + t
        zi = sr // 4
        y0 = (sr % 4) * 16
        return nsrc + (63 + zi + dz) * (_DHW * _DHW) + (63 + y0) * _DHW + 56

    def fire_slabs(t, b0, b1):
        cp0 = pltpu.async_copy(
            src_hbm.at[pl.ds(slab_start(t, 0), _SPAN)],
            b0.at[pl.ds(0, _SPAN)], semp)
        cp1 = pltpu.async_copy(
            src_hbm.at[pl.ds(slab_start(t, 1), _SPAN)],
            b1.at[pl.ds(0, _SPAN)], semp)
        return cp0, cp1

    def wait_slabs(b0, b1):
        pltpu.make_async_copy(
            src_hbm.at[pl.ds(0, _SPAN)], b0.at[pl.ds(0, _SPAN)], semp).wait()
        pltpu.make_async_copy(
            src_hbm.at[pl.ds(0, _SPAN)], b1.at[pl.ds(0, _SPAN)], semp).wait()

    def assemble(b0, b1, rb):
        def yrow(j, c1):
            off0 = j * _DHW + 7
            for gx in range(5):
                off = off0 + gx * _L
                rowv = j * _S + gx * _L + io
                vs = [b0[pl.ds(off, _L)], b0[pl.ds(off + 1, _L)],
                      b0[pl.ds(off + _DHW, _L)], b0[pl.ds(off + _DHW + 1, _L)],
                      b1[pl.ds(off, _L)], b1[pl.ds(off + 1, _L)],
                      b1[pl.ds(off + _DHW, _L)], b1[pl.ds(off + _DHW + 1, _L)]]
                for k in range(8):
                    plsc.store_scatter(rb, [rowv, kcols[k]], vs[k])
            return c1

        lax.fori_loop(0, 16, yrow, 0)

    def fire_rows(t, rb):
        sr = s * _NSR + t
        rbase = trow0 + (sr // 4) * (_S * _S) + (sr % 4) * _RV
        return pltpu.async_copy(
            rb.at[pl.ds(0, _RV)], tabs_hbm.at[pl.ds(rbase, _RV)], semo)

    def wait_rows(rb):
        pltpu.make_async_copy(
            rb.at[pl.ds(0, _RV)], tabs_hbm.at[pl.ds(0, _RV)], semo).wait()

    fire_slabs(0, b0a, b1a)

    def p1_body(u, carry1):
        ta = 2 * u
        tb = 2 * u + 1
        wait_slabs(b0a, b1a)
        fire_slabs(tb, b0b, b1b)

        @pl.when(u > 0)
        def _():
            wait_rows(rba)

        assemble(b0a, b1a, rba)
        fire_rows(ta, rba)

        wait_slabs(b0b, b1b)

        @pl.when(u < (_NSR // 2 - 1))
        def _():
            fire_slabs(ta + 2, b0a, b1a)

        @pl.when(u > 0)
        def _():
            wait_rows(rbb)

        assemble(b0b, b1b, rbb)
        fire_rows(tb, rbb)
        return carry1

    lax.fori_loop(0, _NSR // 2, p1_body, 0)
    wait_rows(rba)
    wait_rows(rbb)
    plsc.subcore_barrier()

    base_pt = c * (_DHW ** 3) + s * _PW
    rbias = trow0 - 63 * (_S * _S + _S + 1)

    def fire_grid(i, gxb, gyb, gzb, sg):
        sl_hbm = pl.ds(base_pt + i * _B, _B)
        pltpu.async_copy(gx_hbm.at[sl_hbm], gxb, sg)
        pltpu.async_copy(gy_hbm.at[sl_hbm], gyb, sg)
        pltpu.async_copy(gz_hbm.at[sl_hbm], gzb, sg)

    def wait_grid(gxb, gyb, gzb, sg):
        for b in (gxb, gyb, gzb):
            pltpu.make_async_copy(gx_hbm.at[pl.ds(0, _B)], b, sg).wait()

    def pass1(gxb, gyb, gzb, fxb, fyb, fzb, idxb):
        def body1(g, c1):
            sl = pl.ds(g * _L, _L)
            px = (gxb[sl] + 1.0) * half
            py = (gyb[sl] + 1.0) * half
            pz = (gzb[sl] + 1.0) * half
            x0 = px.astype(jnp.int32)
            y0 = py.astype(jnp.int32)
            z0 = pz.astype(jnp.int32)
            fxb[sl] = px - x0.astype(jnp.float32)
            fyb[sl] = py - y0.astype(jnp.float32)
            fzb[sl] = pz - z0.astype(jnp.float32)
            idxb[sl] = z0 * (_S * _S) + y0 * _S + x0 + rbias
            return c1

        lax.fori_loop(0, _NG, body1, 0)

    def fire_gather(idxb, vals8):
        pltpu.async_copy(tabs_hbm.at[idxb], vals8, sem)

    def wait_gather(idxb, vals8):
        pltpu.make_async_copy(tabs_hbm.at[idxb], vals8, sem).wait()

    def pass2(vals8, fxb, fyb, fzb, obuf):
        def body2(g, c2):
            sl = pl.ds(g * _L, _L)
            pi = io + g * _L
            v = [plsc.load_gather(vals8, [pi, jnp.full((_L,), k, jnp.int32)])
                 for k in range(8)]
            fx = fxb[sl]
            fy = fyb[sl]
            fz = fzb[sl]
            a00 = v[0] + fx * (v[1] - v[0])
            a01 = v[2] + fx * (v[3] - v[2])
            a10 = v[4] + fx * (v[5] - v[4])
            a11 = v[6] + fx * (v[7] - v[6])
            b0 = a00 + fy * (a01 - a00)
            b1 = a10 + fy * (a11 - a10)
            obuf[sl] = b0 + fz * (b1 - b0)
            return c2

        lax.fori_loop(0, _NG, body2, 0)

    def fire_out(i, obuf, so):
        pltpu.async_copy(obuf, out_hbm.at[pl.ds(base_pt + i * _B, _B)], so)

    def wait_out(obuf, so):
        pltpu.make_async_copy(obuf, out_hbm.at[pl.ds(0, _B)], so).wait()

    fire_grid(0, gx0, gy0, gz0, sg0)
    wait_grid(gx0, gy0, gz0, sg0)
    pass1(gx0, gy0, gz0, fx0, fy0, fz0, ix0)
    fire_gather(ix0, v80)
    fire_grid(1, gx1, gy1, gz1, sg1)

    def p2_body(u, carry2):
        a = 2 * u

        wait_grid(gx1, gy1, gz1, sg1)
        pass1(gx1, gy1, gz1, fx1, fy1, fz1, ix1)
        wait_gather(ix0, v80)
        fire_gather(ix1, v81)

        @pl.when(u > 0)
        def _():
            wait_out(ob0, so0)

        @pl.when(u < _NU - 1)
        def _():
            fire_grid(a + 2, gx0, gy0, gz0, sg0)

        pass2(v80, fx0, fy0, fz0, ob0)
        fire_out(a, ob0, so0)

        @pl.when(u < _NU - 1)
        def _():
            wait_grid(gx0, gy0, gz0, sg0)
            pass1(gx0, gy0, gz0, fx0, fy0, fz0, ix0)

        wait_gather(ix1, v81)

        @pl.when(u < _NU - 1)
        def _():
            fire_gather(ix0, v80)

        @pl.when(u > 0)
        def _():
            wait_out(ob1, so1)

        @pl.when(u < _NU - 1)
        def _():
            fire_grid(a + 3, gx1, gy1, gz1, sg1)

        pass2(v81, fx1, fy1, fz1, ob1)
        fire_out(a + 1, ob1, so1)
        return carry2

    lax.fori_loop(0, _NU, p2_body, 0)
    wait_out(ob0, so0)
    wait_out(ob1, so1)


def kernel(source, affine_grid):
    n, ch, d, h, w = source.shape
    assert (n, ch, d, h, w) == (_N, 1, _DHW, _DHW, _DHW)
    src_flat = jnp.pad(source.reshape(-1), (0, 128))
    gx = affine_grid[..., 0].reshape(-1)
    gy = affine_grid[..., 1].reshape(-1)
    gz = affine_grid[..., 2].reshape(-1)
    out = _sample_kernel(gx, gy, gz, src_flat)
    return out.reshape(n, ch, d, h, w)
